# baseline (speedup 1.0000x reference)
import functools

import jax
import jax.numpy as jnp
from jax import lax
from jax.experimental import pallas as pl
from jax.experimental.pallas import tpu as pltpu
from jax.experimental.pallas import tpu_sc as plsc

B = 16384
F = 26
V = 100000
D = 32
N = B * F

NC = 2
NS = 16
NW = NC * NS
PW = N // NW

VB = 512
NVB = V // VB
VFULL = NVB * VB
VREM = V - VFULL
NU = F * NVB
TMAX = ((NU + NW - 1) // NW + 1) // 2 * 2
ROUNDS = TMAX // 2

CHG = 512
NGG = PW // CHG

CHB = 512
NGB = PW // CHB


def _mesh():
    return plsc.VectorSubcoreMesh(
        core_axis_name="c", subcore_axis_name="s", num_cores=NC, num_subcores=NS
    )


def _sc_transpose(tabT, rem):
    @functools.partial(
        pl.kernel,
        out_type=jax.ShapeDtypeStruct((F * V * D,), jnp.float32),
        mesh=_mesh(),
        compiler_params=pltpu.CompilerParams(use_tc_tiling_on_sc=True,
                                             needs_layout_passes=False),
        scratch_types=[
            pltpu.VMEM((D, VB), jnp.float32),
            pltpu.VMEM((D, VB), jnp.float32),
            pltpu.VMEM((VB * D,), jnp.float32),
            pltpu.VMEM((VB * D,), jnp.float32),
            pltpu.VMEM((VREM * D,), jnp.float32),
            pltpu.SemaphoreType.DMA,
            pltpu.SemaphoreType.DMA,
            pltpu.SemaphoreType.DMA,
            pltpu.SemaphoreType.DMA,
        ],
    )
    def k(tabT_hbm, rem_hbm, lin_out, inb0, inb1, outb0, outb1, rembuf,
          g0, g1, w0, w1):
        wid = lax.axis_index("s") * NC + lax.axis_index("c")
        gsem = [g0, g1]
        wsem = [w0, w1]
        inb = [inb0, inb1]
        outb = [outb0, outb1]

        @pl.when(wid < F)
        def _():
            pltpu.sync_copy(rem_hbm.at[pl.ds(wid * VREM * D, VREM * D)],
                            rembuf)
            pltpu.sync_copy(
                rembuf,
                lin_out.at[pl.ds((wid * V + VFULL) * D, VREM * D)])

        def unit(u):
            f = u // NVB
            vb = u % NVB
            return f, vb

        def issue_in(t, b):
            u = wid + t * NW

            @pl.when(u < NU)
            def _():
                f, vb = unit(u)
                pltpu.async_copy(
                    tabT_hbm.at[pl.ds(f * D, D), pl.ds(vb * VB, VB)],
                    inb[b], gsem[b])

        def wait_in(b):
            pltpu.make_async_copy(
                tabT_hbm.at[pl.ds(0, D), pl.ds(0, VB)], inb[b],
                gsem[b]).wait()

        def issue_out(t, b):
            u = wid + t * NW
            f, vb = unit(u)
            pltpu.async_copy(
                outb[b],
                lin_out.at[pl.ds((f * V + vb * VB) * D, VB * D)], wsem[b])

        def wait_out(b):
            pltpu.make_async_copy(
                outb[b], lin_out.at[pl.ds(0, VB * D)], wsem[b]).wait()

        def transpose(b):
            def body(vv, carry):
                v0 = vv * 16
                vvec = v0 + lax.iota(jnp.int32, 16)
                ovec = vvec * D
                for dd in range(D):
                    vals = plsc.load_gather(
                        inb[b], [jnp.full((16,), dd, jnp.int32), vvec])
                    plsc.store_scatter(outb[b], [ovec + dd], vals)
                return carry
            lax.fori_loop(0, VB // 16, body, 0)

        issue_in(0, 0)
        issue_in(1, 1)

        def round_body(t, carry):
            for b in range(2):
                tt = t * 2 + b
                u = wid + tt * NW

                @pl.when(u < NU)
                def _():
                    wait_in(b)

                    @pl.when(tt >= 2)
                    def _():
                        wait_out(b)
                    transpose(b)
                    issue_out(tt, b)
                    issue_in(tt + 2, b)
            return carry

        lax.fori_loop(0, ROUNDS, round_body, 0)
        for tt in range(TMAX - 4, TMAX):
            u = wid + tt * NW
            u2 = wid + (tt + 2) * NW

            @pl.when((u < NU) & (u2 >= NU))
            def _(tt=tt):
                wait_out(tt % 2)

    return k(tabT, rem)


def _sc_gather(idx2, lin):
    @functools.partial(
        pl.kernel,
        out_type=jax.ShapeDtypeStruct((N, D), jnp.float32),
        mesh=_mesh(),
        compiler_params=pltpu.CompilerParams(use_tc_tiling_on_sc=False,
                                             needs_layout_passes=False),
        scratch_types=[
            pltpu.VMEM((PW,), jnp.int32),
            pltpu.VMEM((2, CHG, D), jnp.float32),
            pltpu.SemaphoreType.DMA,
            pltpu.SemaphoreType.DMA,
            pltpu.SemaphoreType.DMA,
        ],
    )
    def k(idx_hbm, lin_hbm, emb_out, idx_v, rows_v, gsem, wa, wb):
        wid = lax.axis_index("s") * NC + lax.axis_index("c")
        base = wid * PW
        pltpu.sync_copy(idx_hbm.at[wid], idx_v)
        wsem = [wa, wb]
        wdesc = {}
        for g in range(NGG):
            b = g % 2
            if g >= 2:
                wdesc[b].wait()
            pltpu.async_copy(
                lin_hbm.at[idx_v.at[pl.ds(g * CHG, CHG)]],
                rows_v.at[b], gsem).wait()
            wdesc[b] = pltpu.async_copy(
                rows_v.at[b], emb_out.at[pl.ds(base + g * CHG, CHG)],
                wsem[b])
        for b in (0, 1):
            wdesc[b].wait()

    return k(idx2, lin)


def _sc_wide(idx2, lw16):
    @functools.partial(
        pl.kernel,
        out_type=jax.ShapeDtypeStruct((N,), jnp.float32),
        mesh=_mesh(),
        compiler_params=pltpu.CompilerParams(use_tc_tiling_on_sc=False,
                                             needs_layout_passes=False),
        scratch_types=[
            pltpu.VMEM((PW,), jnp.int32),
            pltpu.VMEM((PW,), jnp.int32),
            pltpu.VMEM((CHB, 16), jnp.float32),
            pltpu.VMEM((2, CHB), jnp.float32),
            pltpu.SemaphoreType.DMA,
            pltpu.SemaphoreType.DMA,
            pltpu.SemaphoreType.DMA,
        ],
    )
    def k(idx_hbm, lw16_hbm, wide_out, idx_v, idx16_v, w16_v, wv,
          gsem, wa, wb):
        wid = lax.axis_index("s") * NC + lax.axis_index("c")
        base = wid * PW
        pltpu.sync_copy(idx_hbm.at[wid], idx_v)

        def mk16(i, carry):
            idx16_v[pl.ds(i * 16, 16)] = idx_v[pl.ds(i * 16, 16)] >> 4
            return carry
        lax.fori_loop(0, PW // 16, mk16, 0)

        wsem = [wa, wb]
        wdesc = {}
        for g in range(NGB):
            b = g % 2
            pltpu.async_copy(
                lw16_hbm.at[idx16_v.at[pl.ds(g * CHB, CHB)]],
                w16_v, gsem).wait()
            if g >= 2:
                wdesc[b].wait()

            def sel(s, carry, b=b, g=g):
                lanes = idx_v[pl.ds(g * CHB + s * 16, 16)] & 15
                rvec = s * 16 + lax.iota(jnp.int32, 16)
                wv[b, pl.ds(s * 16, 16)] = plsc.load_gather(
                    w16_v, [rvec, lanes])
                return carry
            lax.fori_loop(0, CHB // 16, sel, 0)
            wdesc[b] = pltpu.async_copy(
                wv.at[b], wide_out.at[pl.ds(base + g * CHB, CHB)], wsem[b])
        for b in (0, 1):
            wdesc[b].wait()

    return k(idx2, lw16)


def _mlp_body(x_ref, wide_ref, w1, b1, w2, b2, w3, b3, wf, bf, o_ref):
    x = x_ref[...]
    h = jnp.maximum(jnp.dot(x, w1[...], preferred_element_type=jnp.float32)
                    + b1[...], 0.0)
    h = jnp.maximum(jnp.dot(h, w2[...], preferred_element_type=jnp.float32)
                    + b2[...], 0.0)
    h = jnp.maximum(jnp.dot(h, w3[...], preferred_element_type=jnp.float32)
                    + b3[...], 0.0)
    deep = jnp.dot(h, wf[...], preferred_element_type=jnp.float32) + bf[...]
    wide = jnp.sum(wide_ref[...], axis=1, keepdims=True)
    o_ref[...] = jax.nn.sigmoid(0.5 * wide + 0.5 * deep)


def _mlp(emb, wide, W1, b1, W2, b2, W3, b3, Wf, bf, block_b=1024):
    nb = B // block_b
    d_in = F * D
    h1, h2, h3 = W1.shape[1], W2.shape[1], W3.shape[1]
    full = lambda shape: pl.BlockSpec(shape, lambda i: (0,) * len(shape))
    return pl.pallas_call(
        _mlp_body,
        grid=(nb,),
        in_specs=[
            pl.BlockSpec((block_b, d_in), lambda i: (i, 0)),
            pl.BlockSpec((block_b, F), lambda i: (i, 0)),
            full((d_in, h1)), full((1, h1)),
            full((h1, h2)), full((1, h2)),
            full((h2, h3)), full((1, h3)),
            full((h3, 1)), full((1, 1)),
        ],
        out_specs=pl.BlockSpec((block_b, 1), lambda i: (i, 0)),
        out_shape=jax.ShapeDtypeStruct((B, 1), jnp.float32),
    )(emb, wide, W1, b1.reshape(1, h1), W2, b2.reshape(1, h2),
      W3, b3.reshape(1, h3), Wf, bf.reshape(1, 1))


def kernel(indices, embed_tables, linear_w, W1, b1, W2, b2, W3, b3, Wf, bf):
    offsets = (jnp.arange(F, dtype=jnp.int32) * V)
    flat_idx = (indices.astype(jnp.int32) + offsets[None, :]).reshape(-1)
    idx2 = flat_idx.reshape(NW, PW)
    tabT = embed_tables.transpose(0, 2, 1).reshape(F * D, V)
    rem = embed_tables[:, VFULL:, :].reshape(F * VREM * D)
    lw16 = linear_w.reshape(F * V // 16, 16)
    lin = _sc_transpose(tabT, rem).reshape(F * V, D)
    emb_flat = _sc_gather(idx2, lin)
    wide_vals = _sc_wide(idx2, lw16)
    emb = emb_flat.reshape(B, F * D)
    wide = wide_vals.reshape(B, F)
    return _mlp(emb, wide, W1, b1, W2, b2, W3, b3, Wf, bf)


# speedup vs baseline: 1.3305x; 1.3305x over previous
import functools

import jax
import jax.numpy as jnp
from jax import lax
from jax.experimental import pallas as pl
from jax.experimental.pallas import tpu as pltpu
from jax.experimental.pallas import tpu_sc as plsc

B = 16384
F = 26
V = 100000
D = 32
N = B * F

NC = 2
NS = 16
NW = NC * NS
PW = N // NW
CH = 128
NCH = PW // CH
G = 8
NG = NCH // G
ROWS = G * CH


def _sc_gather(flat_idx, table, lw16):
    mesh = plsc.VectorSubcoreMesh(
        core_axis_name="c", subcore_axis_name="s", num_cores=NC, num_subcores=NS
    )

    @functools.partial(
        pl.kernel,
        out_type=(
            jax.ShapeDtypeStruct((N, D), jnp.float32),
            jax.ShapeDtypeStruct((N,), jnp.float32),
        ),
        mesh=mesh,
        compiler_params=pltpu.CompilerParams(use_tc_tiling_on_sc=False, needs_layout_passes=False),
        scratch_types=[
            pltpu.VMEM((NCH, CH), jnp.int32),
            pltpu.VMEM((NCH, CH), jnp.int32),
            pltpu.VMEM((2, ROWS, D), jnp.float32),
            pltpu.VMEM((ROWS, 16), jnp.float32),
            pltpu.VMEM((2, ROWS), jnp.float32),
            pltpu.SemaphoreType.DMA,
            pltpu.SemaphoreType.DMA,
            pltpu.SemaphoreType.DMA,
            pltpu.SemaphoreType.DMA,
            pltpu.SemaphoreType.DMA,
            pltpu.SemaphoreType.DMA,
        ],
    )
    def k(idx_hbm, idx16_hbm, table_hbm, lw16_hbm, emb_out, wide_out,
          idx_v, idx16_v, rows_v, w16_v, wv, gsem, gsem2, wa, wb, w2a, w2b):
        wid = lax.axis_index("s") * NC + lax.axis_index("c")
        base = wid * PW
        pltpu.sync_copy(idx_hbm.at[wid], idx_v)
        pltpu.sync_copy(idx16_hbm.at[wid], idx16_v)
        wsems = [wa, wb]
        w2sems = [w2a, w2b]
        wdesc = {}
        for g in range(NG):
            b = g % 2
            if g >= 2:
                wdesc[b][0].wait()
                wdesc[b][1].wait()
            descs = []
            for j in range(G):
                c = g * G + j
                descs.append(pltpu.async_copy(
                    table_hbm.at[idx_v.at[c]],
                    rows_v.at[b, pl.ds(j * CH, CH)], gsem))
                descs.append(pltpu.async_copy(
                    lw16_hbm.at[idx16_v.at[c]],
                    w16_v.at[pl.ds(j * CH, CH)], gsem2))
            for dsc in descs:
                dsc.wait()

            def sel(s, carry, g=g, b=b):
                row = g * G + s // 8
                col = (s % 8) * 16
                lanes = idx_v[row, pl.ds(col, 16)] & 15
                rvec = s * 16 + lax.iota(jnp.int32, 16)
                wv[b, pl.ds(s * 16, 16)] = plsc.load_gather(
                    w16_v, [rvec, lanes])
                return carry

            lax.fori_loop(0, ROWS // 16, sel, 0)
            wdesc[b] = (
                pltpu.async_copy(
                    rows_v.at[b], emb_out.at[pl.ds(base + g * ROWS, ROWS)],
                    wsems[b]),
                pltpu.async_copy(
                    wv.at[b], wide_out.at[pl.ds(base + g * ROWS, ROWS)],
                    w2sems[b]),
            )
        for b in (0, 1):
            wdesc[b][0].wait()
            wdesc[b][1].wait()

    idx3 = flat_idx.reshape(NW, NCH, CH)
    idx16 = (flat_idx >> 4).reshape(NW, NCH, CH)
    return k(idx3, idx16, table, lw16)


def _mlp_body(x_ref, wide_ref, w1, b1, w2, b2, w3, b3, wf, bf, o_ref):
    x = x_ref[...]
    h = jnp.maximum(jnp.dot(x, w1[...], preferred_element_type=jnp.float32)
                    + b1[...], 0.0)
    h = jnp.maximum(jnp.dot(h, w2[...], preferred_element_type=jnp.float32)
                    + b2[...], 0.0)
    h = jnp.maximum(jnp.dot(h, w3[...], preferred_element_type=jnp.float32)
                    + b3[...], 0.0)
    deep = jnp.dot(h, wf[...], preferred_element_type=jnp.float32) + bf[...]
    wide = jnp.sum(wide_ref[...], axis=1, keepdims=True)
    o_ref[...] = jax.nn.sigmoid(0.5 * wide + 0.5 * deep)


def _mlp(emb, wide, W1, b1, W2, b2, W3, b3, Wf, bf, block_b=1024):
    nb = B // block_b
    d_in = F * D
    h1, h2, h3 = W1.shape[1], W2.shape[1], W3.shape[1]
    full = lambda shape: pl.BlockSpec(shape, lambda i: (0,) * len(shape))
    return pl.pallas_call(
        _mlp_body,
        grid=(nb,),
        in_specs=[
            pl.BlockSpec((block_b, d_in), lambda i: (i, 0)),
            pl.BlockSpec((block_b, F), lambda i: (i, 0)),
            full((d_in, h1)), full((1, h1)),
            full((h1, h2)), full((1, h2)),
            full((h2, h3)), full((1, h3)),
            full((h3, 1)), full((1, 1)),
        ],
        out_specs=pl.BlockSpec((block_b, 1), lambda i: (i, 0)),
        out_shape=jax.ShapeDtypeStruct((B, 1), jnp.float32),
    )(emb, wide, W1, b1.reshape(1, h1), W2, b2.reshape(1, h2),
      W3, b3.reshape(1, h3), Wf, bf.reshape(1, 1))


def kernel(indices, embed_tables, linear_w, W1, b1, W2, b2, W3, b3, Wf, bf):
    offsets = (jnp.arange(F, dtype=jnp.int32) * V)
    flat_idx = (indices.astype(jnp.int32) + offsets[None, :]).reshape(-1)
    table = embed_tables.reshape(F * V, D)
    lw16 = linear_w.reshape(F * V // 16, 16)
    emb_flat, wide_vals = _sc_gather(flat_idx, table, lw16)
    emb = emb_flat.reshape(B, F * D)
    wide = wide_vals.reshape(B, F)
    return _mlp(emb, wide, W1, b1, W2, b2, W3, b3, Wf, bf)


# speedup vs baseline: 1.3383x; 1.0059x over previous
import functools

import jax
import jax.numpy as jnp
from jax import lax
from jax.experimental import pallas as pl
from jax.experimental.pallas import tpu as pltpu
from jax.experimental.pallas import tpu_sc as plsc

B = 16384
F = 26
V = 100000
D = 32
N = B * F

NC = 2
NS = 16
NW = NC * NS
PW = N // NW
CH = 128
NCH = PW // CH
G = 8
NG = NCH // G
ROWS = G * CH


def _sc_gather(flat_idx, table, lw16):
    mesh = plsc.VectorSubcoreMesh(
        core_axis_name="c", subcore_axis_name="s", num_cores=NC, num_subcores=NS
    )

    @functools.partial(
        pl.kernel,
        out_type=(
            jax.ShapeDtypeStruct((N, D), jnp.float32),
            jax.ShapeDtypeStruct((N,), jnp.float32),
        ),
        mesh=mesh,
        compiler_params=pltpu.CompilerParams(use_tc_tiling_on_sc=False, needs_layout_passes=False),
        scratch_types=[
            pltpu.VMEM((NCH, CH), jnp.int32),
            pltpu.VMEM((NCH, CH), jnp.int32),
            pltpu.VMEM((2, ROWS, D), jnp.float32),
            pltpu.VMEM((2, ROWS, 16), jnp.float32),
            pltpu.VMEM((2, ROWS), jnp.float32),
            pltpu.SemaphoreType.DMA,
            pltpu.SemaphoreType.DMA,
            pltpu.SemaphoreType.DMA,
            pltpu.SemaphoreType.DMA,
            pltpu.SemaphoreType.DMA,
            pltpu.SemaphoreType.DMA,
            pltpu.SemaphoreType.DMA,
            pltpu.SemaphoreType.DMA,
        ],
    )
    def k(idx_hbm, idx16_hbm, table_hbm, lw16_hbm, emb_out, wide_out,
          idx_v, idx16_v, rows_v, w16_v, wv, gs0, gs1, g2s0, g2s1,
          wa, wb, w2a, w2b):
        wid = lax.axis_index("s") * NC + lax.axis_index("c")
        base = wid * PW
        pltpu.sync_copy(idx_hbm.at[wid], idx_v)
        pltpu.sync_copy(idx16_hbm.at[wid], idx16_v)
        gsems = [gs0, gs1]
        g2sems = [g2s0, g2s1]
        wsems = [wa, wb]
        w2sems = [w2a, w2b]

        def issue(g, b):
            ds = []
            for j in range(G):
                c = g * G + j
                ds.append(pltpu.async_copy(
                    table_hbm.at[idx_v.at[c]],
                    rows_v.at[b, pl.ds(j * CH, CH)], gsems[b]))
                ds.append(pltpu.async_copy(
                    lw16_hbm.at[idx16_v.at[c]],
                    w16_v.at[b, pl.ds(j * CH, CH)], g2sems[b]))
            return ds

        gdesc = {0: issue(0, 0)}
        wdesc = {}
        for g in range(NG):
            b = g % 2
            nb = (g + 1) % 2
            if g + 1 < NG:
                if g >= 1:
                    wdesc[nb][0].wait()
                    wdesc[nb][1].wait()
                gdesc[nb] = issue(g + 1, nb)
            for dsc in gdesc[b]:
                dsc.wait()

            def sel(s, carry, g=g, b=b):
                row = g * G + s // 8
                col = (s % 8) * 16
                lanes = idx_v[row, pl.ds(col, 16)] & 15
                rvec = s * 16 + lax.iota(jnp.int32, 16)
                wv[b, pl.ds(s * 16, 16)] = plsc.load_gather(
                    w16_v.at[b], [rvec, lanes])
                return carry

            lax.fori_loop(0, ROWS // 16, sel, 0)
            wdesc[b] = (
                pltpu.async_copy(
                    rows_v.at[b], emb_out.at[pl.ds(base + g * ROWS, ROWS)],
                    wsems[b]),
                pltpu.async_copy(
                    wv.at[b], wide_out.at[pl.ds(base + g * ROWS, ROWS)],
                    w2sems[b]),
            )
        for b in (0, 1):
            wdesc[b][0].wait()
            wdesc[b][1].wait()

    idx3 = flat_idx.reshape(NW, NCH, CH)
    idx16 = (flat_idx >> 4).reshape(NW, NCH, CH)
    return k(idx3, idx16, table, lw16)


def _mlp_body(x_ref, wide_ref, w1, b1, w2, b2, w3, b3, wf, bf, o_ref):
    x = x_ref[...]
    h = jnp.maximum(jnp.dot(x, w1[...], preferred_element_type=jnp.float32)
                    + b1[...], 0.0)
    h = jnp.maximum(jnp.dot(h, w2[...], preferred_element_type=jnp.float32)
                    + b2[...], 0.0)
    h = jnp.maximum(jnp.dot(h, w3[...], preferred_element_type=jnp.float32)
                    + b3[...], 0.0)
    deep = jnp.dot(h, wf[...], preferred_element_type=jnp.float32) + bf[...]
    wide = jnp.sum(wide_ref[...], axis=1, keepdims=True)
    o_ref[...] = jax.nn.sigmoid(0.5 * wide + 0.5 * deep)


def _mlp(emb, wide, W1, b1, W2, b2, W3, b3, Wf, bf, block_b=1024):
    nb = B // block_b
    d_in = F * D
    h1, h2, h3 = W1.shape[1], W2.shape[1], W3.shape[1]
    full = lambda shape: pl.BlockSpec(shape, lambda i: (0,) * len(shape))
    return pl.pallas_call(
        _mlp_body,
        grid=(nb,),
        in_specs=[
            pl.BlockSpec((block_b, d_in), lambda i: (i, 0)),
            pl.BlockSpec((block_b, F), lambda i: (i, 0)),
            full((d_in, h1)), full((1, h1)),
            full((h1, h2)), full((1, h2)),
            full((h2, h3)), full((1, h3)),
            full((h3, 1)), full((1, 1)),
        ],
        out_specs=pl.BlockSpec((block_b, 1), lambda i: (i, 0)),
        out_shape=jax.ShapeDtypeStruct((B, 1), jnp.float32),
    )(emb, wide, W1, b1.reshape(1, h1), W2, b2.reshape(1, h2),
      W3, b3.reshape(1, h3), Wf, bf.reshape(1, 1))


def kernel(indices, embed_tables, linear_w, W1, b1, W2, b2, W3, b3, Wf, bf):
    offsets = (jnp.arange(F, dtype=jnp.int32) * V)
    flat_idx = (indices.astype(jnp.int32) + offsets[None, :]).reshape(-1)
    table = embed_tables.reshape(F * V, D)
    lw16 = linear_w.reshape(F * V // 16, 16)
    emb_flat, wide_vals = _sc_gather(flat_idx, table, lw16)
    emb = emb_flat.reshape(B, F * D)
    wide = wide_vals.reshape(B, F)
    return _mlp(emb, wide, W1, b1, W2, b2, W3, b3, Wf, bf)
